# Initial kernel scaffold; baseline (speedup 1.0000x reference)
import jax
import jax.numpy as jnp
from jax.experimental import pallas as pl


def kernel(in_embed, out_embed, center_ids, context_ids, negative_ids, vocab_size):
    raise NotImplementedError("write your pallas kernel here")


# speedup vs baseline: 1.1145x; 1.1145x over previous
import functools

import jax
import jax.numpy as jnp
from jax import lax
from jax.experimental import pallas as pl
from jax.experimental.pallas import tpu as pltpu
from jax.experimental.pallas import tpu_sc as plsc


def _sc_gather(V, D, B, R, chunk):
  info = plsc.get_sparse_core_info()
  NC, NS = info.num_cores, info.num_subcores
  NW = NC * NS
  assert B % NW == 0 and R % NW == 0
  cen_per_w = B // NW
  rest_per_w = R // NW
  assert cen_per_w % chunk == 0 and rest_per_w % chunk == 0
  mesh = plsc.VectorSubcoreMesh(core_axis_name="c", subcore_axis_name="s")

  @functools.partial(
      pl.kernel,
      mesh=mesh,
      out_type=[
          jax.ShapeDtypeStruct((B, D), jnp.float32),
          jax.ShapeDtypeStruct((R, D), jnp.float32),
      ],
      scratch_types=[
          pltpu.VMEM((chunk,), jnp.int32),
          pltpu.VMEM((chunk, D), jnp.float32),
          pltpu.SemaphoreType.DMA,
      ],
  )
  def k(in_hbm, out_hbm, cen_ids, rest_ids, cen_out, rest_out, idx_v, rows_v,
        sem):
    wid = lax.axis_index("s") * NC + lax.axis_index("c")

    def do_chunk(ids_hbm, table_hbm, dst_hbm, base):
      pltpu.sync_copy(ids_hbm.at[pl.ds(base, chunk)], idx_v)
      pltpu.async_copy(table_hbm.at[idx_v], rows_v, sem).wait()
      pltpu.sync_copy(rows_v, dst_hbm.at[pl.ds(base, chunk)])

    def cen_body(c, _):
      do_chunk(cen_ids, in_hbm, cen_out, wid * cen_per_w + c * chunk)
      return _

    lax.fori_loop(0, cen_per_w // chunk, cen_body, 0)

    def rest_body(c, _):
      do_chunk(rest_ids, out_hbm, rest_out, wid * rest_per_w + c * chunk)
      return _

    lax.fori_loop(0, rest_per_w // chunk, rest_body, 0)

  return k


def _tc_loss_body(cen_ref, rest_ref, acc_ref, *, n_pos, n_neg):
  i = pl.program_id(0)

  @pl.when(i == 0)
  def _():
    acc_ref[...] = jnp.zeros_like(acc_ref)

  cen = cen_ref[...]
  rest = rest_ref[...]
  scores = jnp.sum(cen[None, :, :] * rest, axis=-1)
  pos = scores[0]
  neg = scores[1:]
  pos_terms = -jnp.log(jax.nn.sigmoid(pos) + 1e-08)
  neg_terms = -jnp.log(jax.nn.sigmoid(-neg) + 1e-08)
  acc_ref[...] += jnp.full((1, 1), jnp.sum(pos_terms) / n_pos +
                           jnp.sum(neg_terms) / n_neg)


def kernel(in_embed, out_embed, center_ids, context_ids, negative_ids,
           vocab_size):
  V, D = in_embed.shape
  B = center_ids.shape[0]
  K = negative_ids.shape[0]
  R = (K + 1) * B

  rest_ids = jnp.concatenate([context_ids, negative_ids.reshape(-1)])
  cen_rows, rest_rows = _sc_gather(V, D, B, R, chunk=512)(
      in_embed, out_embed, center_ids, rest_ids)
  rest_rows = rest_rows.reshape(K + 1, B, D)

  Nb = 1024
  grid = (B // Nb,)
  acc = pl.pallas_call(
      functools.partial(_tc_loss_body, n_pos=float(B), n_neg=float(K * B)),
      grid=grid,
      in_specs=[
          pl.BlockSpec((Nb, D), lambda i: (i, 0)),
          pl.BlockSpec((K + 1, Nb, D), lambda i: (0, i, 0)),
      ],
      out_specs=pl.BlockSpec((1, 1), lambda i: (0, 0)),
      out_shape=jax.ShapeDtypeStruct((1, 1), jnp.float32),
  )(cen_rows, rest_rows)
  return acc[0, 0]
